# baseline (speedup 1.0000x reference)
import functools

import jax
import jax.numpy as jnp
from jax import lax
from jax.experimental import pallas as pl
from jax.experimental.pallas import tpu as pltpu
from jax.experimental.pallas import tpu_sc as plsc

_NUM_CORES = 2
_NUM_SUBCORES = 16
_LANES = 16
_NW = _NUM_CORES * _NUM_SUBCORES


@functools.partial(jax.jit, static_argnums=(0, 1, 2))
def _pos_embed_sc(B, H, W, row_embed, col_embed):
    D = row_embed.shape[1]
    C = 2 * D
    CPW = D // _NW
    NJ = W // _LANES
    PLANE = H * W
    mesh = plsc.VectorSubcoreMesh(core_axis_name="c", subcore_axis_name="s")

    row_flat = row_embed[:H].reshape(H * D)
    col_flat = col_embed[:H].reshape(H * D)

    @functools.partial(
        pl.kernel,
        out_type=jax.ShapeDtypeStruct((B * C * PLANE,), jnp.float32),
        mesh=mesh,
        compiler_params=pltpu.CompilerParams(needs_layout_passes=False),
        scratch_types=[
            pltpu.VMEM((H * D,), jnp.float32),
            pltpu.VMEM((H * D,), jnp.float32),
            pltpu.VMEM((2 * CPW * PLANE,), jnp.float32),
            pltpu.SemaphoreType.DMA,
        ],
    )
    def k(row_hbm, col_hbm, out_hbm, tabc, tabr, buf, sem):
        wid = lax.axis_index("s") * _NUM_CORES + lax.axis_index("c")
        cbase = wid * CPW

        pltpu.sync_copy(col_hbm, tabc)
        pltpu.sync_copy(row_hbm, tabr)

        base_ids = lax.iota(jnp.int32, _LANES)

        for cl in range(CPW):
            ch = cbase + cl
            vecs = [
                plsc.load_gather(tabc, [(base_ids + kj * _LANES) * D + ch])
                for kj in range(NJ)
            ]

            def col_body(i, _, cl=cl, vecs=vecs):
                for kj in range(NJ):
                    buf[pl.ds(cl * PLANE + i * W + kj * _LANES, _LANES)] = (
                        vecs[kj])
                return 0

            lax.fori_loop(0, H, col_body, 0)

        def row_body(i, _):
            v = tabr[pl.ds(i * D + cbase, _LANES)]
            for cl in range(CPW):
                s = jnp.full((_LANES,), v[cl], jnp.float32)
                for kj in range(NJ):
                    buf[pl.ds((CPW + cl) * PLANE + i * W + kj * _LANES,
                              _LANES)] = s
            return 0

        lax.fori_loop(0, H, row_body, 0)

        chunk = CPW * PLANE
        copies = []
        for b in range(B):
            copies.append(pltpu.async_copy(
                buf.at[pl.ds(0, chunk)],
                out_hbm.at[pl.ds((b * C + cbase) * PLANE, chunk)], sem))
            copies.append(pltpu.async_copy(
                buf.at[pl.ds(chunk, chunk)],
                out_hbm.at[pl.ds((b * C + D + cbase) * PLANE, chunk)], sem))
        for c in copies:
            c.wait()

    return k(row_flat, col_flat).reshape(B, C, H, W)


def kernel(x, row_embed, col_embed):
    B = x.shape[0]
    H, W = x.shape[-2], x.shape[-1]
    return _pos_embed_sc(B, H, W, row_embed, col_embed)


# speedup vs baseline: 1.8195x; 1.8195x over previous
import functools

import jax
import jax.numpy as jnp
from jax.experimental import pallas as pl
from jax.experimental.pallas import tpu as pltpu


@functools.partial(jax.jit, static_argnums=(0, 1, 2))
def _pos_embed_tc(B, H, W, row_embed, col_embed):
    D = row_embed.shape[1]
    C = 2 * D

    def body(row_ref, col_ref, o_ref):
        colT = col_ref[:H, :].T
        rowT = row_ref[:W, :].T
        o_ref[0, :D] = jnp.broadcast_to(colT[:, None, :], (D, H, W))
        o_ref[0, D:] = jnp.broadcast_to(rowT[:, :, None], (D, H, W))

    return pl.pallas_call(
        body,
        grid=(B,),
        in_specs=[
            pl.BlockSpec(row_embed.shape, lambda b: (0, 0)),
            pl.BlockSpec(col_embed.shape, lambda b: (0, 0)),
        ],
        out_specs=pl.BlockSpec((1, C, H, W), lambda b: (b, 0, 0, 0)),
        out_shape=jax.ShapeDtypeStruct((B, C, H, W), jnp.float32),
    )(row_embed, col_embed)


def kernel(x, row_embed, col_embed):
    B = x.shape[0]
    H, W = x.shape[-2], x.shape[-1]
    return _pos_embed_tc(B, H, W, row_embed, col_embed)


# speedup vs baseline: 4.5072x; 2.4771x over previous
import functools

import jax
import jax.numpy as jnp
from jax import lax
from jax.experimental import pallas as pl
from jax.experimental.pallas import tpu as pltpu


@functools.partial(jax.jit, static_argnums=(0, 1, 2))
def _pos_embed_tc(B, H, W, row_embed, col_embed):
    D = row_embed.shape[1]
    C = 2 * D
    P = H * W

    def body(row_ref, col_ref, o_ref):
        p_lane = lax.broadcasted_iota(jnp.int32, (H, P), 1)
        sub = lax.broadcasted_iota(jnp.int32, (H, P), 0)
        s_col = (p_lane % W == sub).astype(jnp.float32)
        s_row = (p_lane // W == sub).astype(jnp.float32)
        dn = (((0,), (0,)), ((), ()))
        o_ref[0, :D] = lax.dot_general(
            col_ref[:W, :], s_col, dn, preferred_element_type=jnp.float32)
        o_ref[0, D:] = lax.dot_general(
            row_ref[:H, :], s_row, dn, preferred_element_type=jnp.float32)

    out = pl.pallas_call(
        body,
        grid=(B,),
        in_specs=[
            pl.BlockSpec(row_embed.shape, lambda b: (0, 0)),
            pl.BlockSpec(col_embed.shape, lambda b: (0, 0)),
        ],
        out_specs=pl.BlockSpec((1, C, P), lambda b: (b, 0, 0)),
        out_shape=jax.ShapeDtypeStruct((B, C, P), jnp.float32),
    )(row_embed, col_embed)
    return out.reshape(B, C, H, W)


def kernel(x, row_embed, col_embed):
    B = x.shape[0]
    H, W = x.shape[-2], x.shape[-1]
    return _pos_embed_tc(B, H, W, row_embed, col_embed)


# speedup vs baseline: 13.9266x; 3.0899x over previous
import functools

import jax
import jax.numpy as jnp
from jax.experimental import pallas as pl


@functools.partial(jax.jit, static_argnums=(0, 1, 2))
def _pos_embed_tc(B, H, W, row_embed, col_embed):
    D = row_embed.shape[1]

    def body(row_ref, col_ref, o_ref):
        col = col_ref[:W, :]
        row = row_ref[:H, :]
        o_ref[0, :, :, :D] = jnp.broadcast_to(col[None, :, :], (H, W, D))
        o_ref[0, :, :, D:] = jnp.broadcast_to(row[:, None, :], (H, W, D))

    out = pl.pallas_call(
        body,
        grid=(B,),
        in_specs=[
            pl.BlockSpec(row_embed.shape, lambda b: (0, 0)),
            pl.BlockSpec(col_embed.shape, lambda b: (0, 0)),
        ],
        out_specs=pl.BlockSpec((1, H, W, 2 * D), lambda b: (b, 0, 0, 0)),
        out_shape=jax.ShapeDtypeStruct((B, H, W, 2 * D), jnp.float32),
    )(row_embed, col_embed)
    return jnp.transpose(out, (0, 3, 1, 2))


def kernel(x, row_embed, col_embed):
    B = x.shape[0]
    H, W = x.shape[-2], x.shape[-1]
    return _pos_embed_tc(B, H, W, row_embed, col_embed)
